# baseline (speedup 1.0000x reference)
import functools

import jax
import jax.numpy as jnp
import numpy as np
from jax import lax
from jax.experimental import pallas as pl
from jax.experimental.pallas import tpu as pltpu
from jax.experimental.pallas import tpu_sc as plsc

VOCAB = 1000000
EMB = 64
HID = 128
NCLS = 50
B = 4096
L = 200

NC = 2
NS = 16
NW = NC * NS
ROWS_PER_W = B // NW
CH0, CH1 = 128, L - 128
NBUF = 4

COL_PERM = np.concatenate([
    np.arange(0, 32, 2), np.arange(1, 32, 2),
    np.arange(32, 64, 2), np.arange(33, 64, 2),
])


def _sc_pool_kernel(x_hbm, emb_hbm, out_hbm,
                    idx_v, b0, b1, b2, b3, out_v, s0, s1, s2, s3):
    bufs = (b0, b1, b2, b3)
    sems = (s0, s1, s2, s3)
    wid = lax.axis_index("s") * NC + lax.axis_index("c")
    base = wid * ROWS_PER_W
    pltpu.sync_copy(x_hbm.at[pl.ds(base, ROWS_PER_W)], idx_v)

    def fire(r, buf, sem):
        pltpu.async_copy(
            emb_hbm.at[idx_v.at[r, pl.ds(0, CH0)]], buf.at[pl.ds(0, CH0)], sem)
        pltpu.async_copy(
            emb_hbm.at[idx_v.at[r, pl.ds(CH0, CH1)]], buf.at[pl.ds(CH0, CH1)], sem)

    def drain(buf, sem):
        pltpu.make_async_copy(emb_hbm.at[pl.ds(0, L)], buf, sem).wait()

    def reduce_row(buf, r):
        def rbody(lo, accs):
            a0, a1, a2, a3 = accs
            for j in range(8):
                l = lo * 8 + j
                lo16 = buf[l, pl.ds(0, 32)]
                hi16 = buf[l, pl.ds(32, 32)]
                ev0, od0 = plsc.unpack(lo16, format=plsc.PackFormat.INTERLEAVED)
                ev1, od1 = plsc.unpack(hi16, format=plsc.PackFormat.INTERLEAVED)
                a0 = a0 + ev0
                a1 = a1 + od0
                a2 = a2 + ev1
                a3 = a3 + od1
            return a0, a1, a2, a3
        z = jnp.zeros((16,), jnp.float32)
        a0, a1, a2, a3 = lax.fori_loop(0, L // 8, rbody, (z, z, z, z))
        out_v[r, pl.ds(0, 16)] = a0
        out_v[r, pl.ds(16, 16)] = a1
        out_v[r, pl.ds(32, 16)] = a2
        out_v[r, pl.ds(48, 16)] = a3

    for b in range(NBUF):
        fire(b, bufs[b], sems[b])

    def gbody(g, carry):
        for b in range(NBUF):
            r = g * NBUF + b
            drain(bufs[b], sems[b])
            reduce_row(bufs[b], r)
            nxt = r + NBUF

            @pl.when(nxt < ROWS_PER_W)
            def _():
                fire(nxt, bufs[b], sems[b])
        return carry

    lax.fori_loop(0, ROWS_PER_W // NBUF, gbody, 0)
    pltpu.sync_copy(out_v, out_hbm.at[pl.ds(base, ROWS_PER_W)])


_sc_pool = functools.partial(
    pl.kernel,
    mesh=plsc.VectorSubcoreMesh(core_axis_name="c", subcore_axis_name="s"),
    compiler_params=pltpu.CompilerParams(
        use_tc_tiling_on_sc=False, needs_layout_passes=False),
    out_type=jax.ShapeDtypeStruct((B, EMB), jnp.float32),
    scratch_types=[
        pltpu.VMEM((ROWS_PER_W, L), jnp.int32),
        pltpu.VMEM((L, EMB), jnp.bfloat16),
        pltpu.VMEM((L, EMB), jnp.bfloat16),
        pltpu.VMEM((L, EMB), jnp.bfloat16),
        pltpu.VMEM((L, EMB), jnp.bfloat16),
        pltpu.VMEM((ROWS_PER_W, EMB), jnp.float32),
        pltpu.SemaphoreType.DMA,
        pltpu.SemaphoreType.DMA,
        pltpu.SemaphoreType.DMA,
        pltpu.SemaphoreType.DMA,
    ],
)(_sc_pool_kernel)


def _mlp_kernel(p_ref, w1_ref, b1_ref, w2_ref, b2_ref, o_ref):
    p = p_ref[...] * (1.0 / L)
    h = jnp.dot(p, w1_ref[...], preferred_element_type=jnp.float32) + b1_ref[...]
    h = jnp.maximum(h, 0.0)
    o_ref[...] = jnp.dot(h, w2_ref[...], preferred_element_type=jnp.float32) + b2_ref[...]


def kernel(x, emb, W1, b1, W2, b2):
    emb16 = emb.astype(jnp.bfloat16)
    pooled_sum = _sc_pool(x, emb16)
    W1p = W1[COL_PERM, :]
    out = pl.pallas_call(
        _mlp_kernel,
        out_shape=jax.ShapeDtypeStruct((B, NCLS), jnp.float32),
    )(pooled_sum, W1p, b1.reshape(1, HID), W2, b2.reshape(1, NCLS))
    return out


# speedup vs baseline: 2.0328x; 2.0328x over previous
import functools

import jax
import jax.numpy as jnp
from jax import lax
from jax.experimental import pallas as pl
from jax.experimental.pallas import tpu as pltpu
from jax.experimental.pallas import tpu_sc as plsc

VOCAB = 1000000
EMB = 64
HID = 128
NCLS = 50
B = 4096
L = 200
WIDE = 128

VBLK = 8192
HALF = VBLK // 2
NBLK = (VOCAB + VBLK - 1) // VBLK
TROWS = NBLK * HALF

NC = 2
NS = 16
NW = NC * NS
ROWS_PER_W = B // NW
CH0, CH1 = 128, L - 128
NBUF = 3
NCHUNK = 13


def _pack_kernel(in_ref, o_ref):
    t0 = jnp.transpose(in_ref[:, 0:HALF])
    t1 = jnp.transpose(in_ref[:, HALF:VBLK])
    o_ref[...] = jnp.concatenate([t0, t1], axis=1)


def _pack_table(embT):
    return pl.pallas_call(
        _pack_kernel,
        grid=(NBLK,),
        in_specs=[pl.BlockSpec((EMB, VBLK), lambda i: (0, i))],
        out_specs=pl.BlockSpec((HALF, WIDE), lambda i: (i, 0)),
        out_shape=jax.ShapeDtypeStruct((TROWS, WIDE), jnp.float32),
    )(embT)


def _sc_pool_kernel(x_hbm, emb_hbm, out_hbm,
                    idx_v, b0, b1, b2, c0, c1, c2, ne_s, out_v, s0, s1, s2):
    bufs = (b0, b1, b2)
    combs = (c0, c1, c2)
    sems = (s0, s1, s2)
    wid = lax.axis_index("s") * NC + lax.axis_index("c")
    base = wid * ROWS_PER_W
    pltpu.sync_copy(x_hbm.at[pl.ds(base, ROWS_PER_W)], idx_v)

    lane = lax.iota(jnp.int32, 16)

    def compact(r, comb):
        ne = jnp.int32(0)
        no = jnp.int32(0)
        for c in range(NCHUNK):
            off = 184 if c == NCHUNK - 1 else c * 16
            v = idx_v[r, pl.ds(off, 16)]
            valid = lane >= 8 if c == NCHUNK - 1 else lane >= 0
            loc = jnp.bitwise_and(v, VBLK - 1)
            row = jnp.bitwise_or(
                jnp.left_shift(jnp.right_shift(v, 13), 12),
                jnp.bitwise_and(loc, HALF - 1))
            is_hi = jnp.right_shift(loc, 12)
            mask_e = jnp.logical_and(is_hi == 0, valid)
            mask_o = jnp.logical_and(is_hi == 1, valid)
            ce = plsc.cumsum(jnp.where(mask_e, 1, 0).astype(jnp.int32))
            co = plsc.cumsum(jnp.where(mask_o, 1, 0).astype(jnp.int32))
            plsc.store_scatter(comb, [ne + ce - 1], row, mask=mask_e)
            plsc.store_scatter(comb, [jnp.int32(L - 1) - (no + co - 1)], row,
                               mask=mask_o)
            ne = ne + jnp.sum(jnp.where(mask_e, 1, 0).astype(jnp.int32))
            no = no + jnp.sum(jnp.where(mask_o, 1, 0).astype(jnp.int32))
        ne_s[r] = ne

    def fire(r, buf, sem, comb):
        compact(r, comb)
        pltpu.async_copy(
            emb_hbm.at[comb.at[pl.ds(0, CH0)]], buf.at[pl.ds(0, CH0)], sem)
        pltpu.async_copy(
            emb_hbm.at[comb.at[pl.ds(CH0, CH1)]], buf.at[pl.ds(CH0, CH1)], sem)

    def drain(buf, sem):
        pltpu.make_async_copy(emb_hbm.at[pl.ds(0, L)], buf, sem).wait()

    def reduce_row(buf, r):
        ne = ne_s[r]

        def phase(lo_start, lo_end, lane0, accs):
            def body(l, accs):
                a0, a1, a2, a3 = accs
                a0 = a0 + buf[l, pl.ds(lane0 + 0, 16)]
                a1 = a1 + buf[l, pl.ds(lane0 + 16, 16)]
                a2 = a2 + buf[l, pl.ds(lane0 + 32, 16)]
                a3 = a3 + buf[l, pl.ds(lane0 + 48, 16)]
                return a0, a1, a2, a3
            return lax.fori_loop(lo_start, lo_end, body, accs)

        z = jnp.zeros((16,), jnp.float32)
        accs = phase(0, ne, 0, (z, z, z, z))
        a0, a1, a2, a3 = phase(ne, L, EMB, accs)
        out_v[r, pl.ds(0, 16)] = a0
        out_v[r, pl.ds(16, 16)] = a1
        out_v[r, pl.ds(32, 16)] = a2
        out_v[r, pl.ds(48, 16)] = a3

    for b in range(NBUF):
        fire(b, bufs[b], sems[b], combs[b])

    def gbody(g, carry):
        for b in range(NBUF):
            r = g * NBUF + b
            drain(bufs[b], sems[b])
            reduce_row(bufs[b], r)
            nxt = r + NBUF

            @pl.when(nxt < ROWS_PER_W)
            def _():
                fire(nxt, bufs[b], sems[b], combs[b])
        return carry

    main = (ROWS_PER_W // NBUF) * NBUF
    lax.fori_loop(0, ROWS_PER_W // NBUF, gbody, 0)
    for r in range(main, ROWS_PER_W):
        b = r % NBUF
        drain(bufs[b], sems[b])
        reduce_row(bufs[b], r)
    pltpu.sync_copy(out_v, out_hbm.at[pl.ds(base, ROWS_PER_W)])


_sc_pool = functools.partial(
    pl.kernel,
    mesh=plsc.VectorSubcoreMesh(core_axis_name="c", subcore_axis_name="s"),
    compiler_params=pltpu.CompilerParams(
        use_tc_tiling_on_sc=False, needs_layout_passes=False),
    out_type=jax.ShapeDtypeStruct((B, EMB), jnp.float32),
    scratch_types=[
        pltpu.VMEM((ROWS_PER_W, L), jnp.int32),
        pltpu.VMEM((L, WIDE), jnp.float32),
        pltpu.VMEM((L, WIDE), jnp.float32),
        pltpu.VMEM((L, WIDE), jnp.float32),
        pltpu.VMEM((L,), jnp.int32),
        pltpu.VMEM((L,), jnp.int32),
        pltpu.VMEM((L,), jnp.int32),
        pltpu.SMEM((ROWS_PER_W,), jnp.int32),
        pltpu.VMEM((ROWS_PER_W, EMB), jnp.float32),
        pltpu.SemaphoreType.DMA,
        pltpu.SemaphoreType.DMA,
        pltpu.SemaphoreType.DMA,
    ],
)(_sc_pool_kernel)


def _mlp_kernel(p_ref, w1_ref, b1_ref, w2_ref, b2_ref, o_ref):
    p = p_ref[...] * (1.0 / L)
    h = jnp.dot(p, w1_ref[...], preferred_element_type=jnp.float32) + b1_ref[...]
    h = jnp.maximum(h, 0.0)
    o_ref[...] = jnp.dot(h, w2_ref[...], preferred_element_type=jnp.float32) + b2_ref[...]


def kernel(x, emb, W1, b1, W2, b2):
    table = _pack_table(emb.T)
    pooled_sum = _sc_pool(x, table)
    out = pl.pallas_call(
        _mlp_kernel,
        out_shape=jax.ShapeDtypeStruct((B, NCLS), jnp.float32),
    )(pooled_sum, W1, b1.reshape(1, HID), W2, b2.reshape(1, NCLS))
    return out


# speedup vs baseline: 2.1578x; 1.0615x over previous
import functools

import jax
import jax.numpy as jnp
from jax import lax
from jax.experimental import pallas as pl
from jax.experimental.pallas import tpu as pltpu
from jax.experimental.pallas import tpu_sc as plsc

VOCAB = 1000000
EMB = 64
HID = 128
NCLS = 50
B = 4096
L = 200
WIDE = 128

VBLK = 8192
HALF = VBLK // 2
NBLK = (VOCAB + VBLK - 1) // VBLK
TROWS = NBLK * HALF

NC = 2
NS = 16
NW = NC * NS
ROWS_PER_W = B // NW
CH0, CH1 = 128, L - 128
NBUF = 3
NCHUNK = 13


def _pack_kernel(in_ref, o_ref):
    row_i = lax.broadcasted_iota(jnp.int32, (EMB, WIDE), 0)
    col_i = lax.broadcasted_iota(jnp.int32, (EMB, WIDE), 1)
    eye_lo = (row_i == col_i).astype(jnp.float32)
    eye_hi = (row_i + EMB == col_i).astype(jnp.float32)
    dn = (((0,), (0,)), ((), ()))
    t0 = lax.dot_general(in_ref[:, 0:HALF], eye_lo, dn,
                         preferred_element_type=jnp.float32)
    t1 = lax.dot_general(in_ref[:, HALF:VBLK], eye_hi, dn,
                         preferred_element_type=jnp.float32)
    o_ref[...] = t0 + t1


def _pack_table(embT):
    return pl.pallas_call(
        _pack_kernel,
        grid=(NBLK,),
        in_specs=[pl.BlockSpec((EMB, VBLK), lambda i: (0, i))],
        out_specs=pl.BlockSpec((HALF, WIDE), lambda i: (i, 0)),
        out_shape=jax.ShapeDtypeStruct((TROWS, WIDE), jnp.float32),
    )(embT)


def _sc_pool_kernel(x_hbm, emb_hbm, out_hbm,
                    idx_v, b0, b1, b2, c0, c1, c2, ne_s, out_v, s0, s1, s2):
    bufs = (b0, b1, b2)
    combs = (c0, c1, c2)
    sems = (s0, s1, s2)
    wid = lax.axis_index("s") * NC + lax.axis_index("c")
    base = wid * ROWS_PER_W
    pltpu.sync_copy(x_hbm.at[pl.ds(base, ROWS_PER_W)], idx_v)

    lane = lax.iota(jnp.int32, 16)

    def compact(r, comb):
        ne = jnp.int32(0)
        no = jnp.int32(0)
        for c in range(NCHUNK):
            off = 184 if c == NCHUNK - 1 else c * 16
            v = idx_v[r, pl.ds(off, 16)]
            valid = lane >= 8 if c == NCHUNK - 1 else lane >= 0
            loc = jnp.bitwise_and(v, VBLK - 1)
            row = jnp.bitwise_or(
                jnp.left_shift(jnp.right_shift(v, 13), 12),
                jnp.bitwise_and(loc, HALF - 1))
            is_hi = jnp.right_shift(loc, 12)
            mask_e = jnp.logical_and(is_hi == 0, valid)
            mask_o = jnp.logical_and(is_hi == 1, valid)
            ce = plsc.cumsum(jnp.where(mask_e, 1, 0).astype(jnp.int32))
            co = plsc.cumsum(jnp.where(mask_o, 1, 0).astype(jnp.int32))
            plsc.store_scatter(comb, [ne + ce - 1], row, mask=mask_e)
            plsc.store_scatter(comb, [jnp.int32(L - 1) - (no + co - 1)], row,
                               mask=mask_o)
            ne = ne + jnp.sum(jnp.where(mask_e, 1, 0).astype(jnp.int32))
            no = no + jnp.sum(jnp.where(mask_o, 1, 0).astype(jnp.int32))
        ne_s[r] = ne

    def fire(r, buf, sem, comb):
        compact(r, comb)
        pltpu.async_copy(
            emb_hbm.at[comb.at[pl.ds(0, CH0)]], buf.at[pl.ds(0, CH0)], sem)
        pltpu.async_copy(
            emb_hbm.at[comb.at[pl.ds(CH0, CH1)]], buf.at[pl.ds(CH0, CH1)], sem)

    def drain(buf, sem):
        pltpu.make_async_copy(emb_hbm.at[pl.ds(0, L)], buf, sem).wait()

    def reduce_row(buf, r):
        ne = ne_s[r]

        def phase(lo_start, lo_end, lane0, accs):
            def body(l, accs):
                a0, a1, a2, a3 = accs
                a0 = a0 + buf[l, pl.ds(lane0 + 0, 16)]
                a1 = a1 + buf[l, pl.ds(lane0 + 16, 16)]
                a2 = a2 + buf[l, pl.ds(lane0 + 32, 16)]
                a3 = a3 + buf[l, pl.ds(lane0 + 48, 16)]
                return a0, a1, a2, a3
            return lax.fori_loop(lo_start, lo_end, body, accs)

        z = jnp.zeros((16,), jnp.float32)
        accs = phase(0, ne, 0, (z, z, z, z))
        a0, a1, a2, a3 = phase(ne, L, EMB, accs)
        out_v[r, pl.ds(0, 16)] = a0
        out_v[r, pl.ds(16, 16)] = a1
        out_v[r, pl.ds(32, 16)] = a2
        out_v[r, pl.ds(48, 16)] = a3

    for b in range(NBUF):
        fire(b, bufs[b], sems[b], combs[b])

    def gbody(g, carry):
        for b in range(NBUF):
            r = g * NBUF + b
            drain(bufs[b], sems[b])
            reduce_row(bufs[b], r)
            nxt = r + NBUF

            @pl.when(nxt < ROWS_PER_W)
            def _():
                fire(nxt, bufs[b], sems[b], combs[b])
        return carry

    main = (ROWS_PER_W // NBUF) * NBUF
    lax.fori_loop(0, ROWS_PER_W // NBUF, gbody, 0)
    for r in range(main, ROWS_PER_W):
        b = r % NBUF
        drain(bufs[b], sems[b])
        reduce_row(bufs[b], r)
    pltpu.sync_copy(out_v, out_hbm.at[pl.ds(base, ROWS_PER_W)])


_sc_pool = functools.partial(
    pl.kernel,
    mesh=plsc.VectorSubcoreMesh(core_axis_name="c", subcore_axis_name="s"),
    compiler_params=pltpu.CompilerParams(
        use_tc_tiling_on_sc=False, needs_layout_passes=False),
    out_type=jax.ShapeDtypeStruct((B, EMB), jnp.float32),
    scratch_types=[
        pltpu.VMEM((ROWS_PER_W, L), jnp.int32),
        pltpu.VMEM((L, WIDE), jnp.float32),
        pltpu.VMEM((L, WIDE), jnp.float32),
        pltpu.VMEM((L, WIDE), jnp.float32),
        pltpu.VMEM((L,), jnp.int32),
        pltpu.VMEM((L,), jnp.int32),
        pltpu.VMEM((L,), jnp.int32),
        pltpu.SMEM((ROWS_PER_W,), jnp.int32),
        pltpu.VMEM((ROWS_PER_W, EMB), jnp.float32),
        pltpu.SemaphoreType.DMA,
        pltpu.SemaphoreType.DMA,
        pltpu.SemaphoreType.DMA,
    ],
)(_sc_pool_kernel)


def _mlp_kernel(p_ref, w1_ref, b1_ref, w2_ref, b2_ref, o_ref):
    p = p_ref[...] * (1.0 / L)
    h = jnp.dot(p, w1_ref[...], preferred_element_type=jnp.float32) + b1_ref[...]
    h = jnp.maximum(h, 0.0)
    o_ref[...] = jnp.dot(h, w2_ref[...], preferred_element_type=jnp.float32) + b2_ref[...]


def kernel(x, emb, W1, b1, W2, b2):
    table = _pack_table(emb.T)
    pooled_sum = _sc_pool(x, table)
    out = pl.pallas_call(
        _mlp_kernel,
        out_shape=jax.ShapeDtypeStruct((B, NCLS), jnp.float32),
    )(pooled_sum, W1, b1.reshape(1, HID), W2, b2.reshape(1, NCLS))
    return out


# speedup vs baseline: 2.3793x; 1.1026x over previous
import functools

import jax
import jax.numpy as jnp
from jax import lax
from jax.experimental import pallas as pl
from jax.experimental.pallas import tpu as pltpu
from jax.experimental.pallas import tpu_sc as plsc

VOCAB = 1000000
EMB = 64
HID = 128
NCLS = 50
B = 4096
L = 200
WIDE = 128

VBLK = 16384
HALF = VBLK // 2
NBLK = (VOCAB + VBLK - 1) // VBLK
TROWS = NBLK * HALF

NC = 2
NS = 16
NW = NC * NS
ROWS_PER_W = B // NW
CH0, CH1 = 128, L - 128
NBUF = 3
NCHUNK = 13


def _pack_kernel(in_ref, o_ref):
    row_i = lax.broadcasted_iota(jnp.int32, (EMB, WIDE), 0)
    col_i = lax.broadcasted_iota(jnp.int32, (EMB, WIDE), 1)
    eye_lo = (row_i == col_i).astype(jnp.float32)
    eye_hi = (row_i + EMB == col_i).astype(jnp.float32)
    dn = (((0,), (0,)), ((), ()))
    t0 = lax.dot_general(in_ref[:, 0:HALF], eye_lo, dn,
                         preferred_element_type=jnp.float32)
    t1 = lax.dot_general(in_ref[:, HALF:VBLK], eye_hi, dn,
                         preferred_element_type=jnp.float32)
    o_ref[...] = t0 + t1


def _pack_table(embT):
    return pl.pallas_call(
        _pack_kernel,
        grid=(NBLK,),
        in_specs=[pl.BlockSpec((EMB, VBLK), lambda i: (0, i))],
        out_specs=pl.BlockSpec((HALF, WIDE), lambda i: (i, 0)),
        out_shape=jax.ShapeDtypeStruct((TROWS, WIDE), jnp.float32),
    )(embT)


def _sc_pool_kernel(x_hbm, emb_hbm, out_hbm,
                    idx_v, b0, b1, b2, c0, c1, c2, ne_s, out_v, s0, s1, s2):
    bufs = (b0, b1, b2)
    combs = (c0, c1, c2)
    sems = (s0, s1, s2)
    wid = lax.axis_index("s") * NC + lax.axis_index("c")
    base = wid * ROWS_PER_W
    pltpu.sync_copy(x_hbm.at[pl.ds(base, ROWS_PER_W)], idx_v)

    lane = lax.iota(jnp.int32, 16)

    def compact(r, comb):
        ne = jnp.int32(0)
        no = jnp.int32(0)
        for c in range(NCHUNK):
            off = 184 if c == NCHUNK - 1 else c * 16
            v = idx_v[r, pl.ds(off, 16)]
            valid = lane >= 8 if c == NCHUNK - 1 else lane >= 0
            loc = jnp.bitwise_and(v, VBLK - 1)
            row = jnp.bitwise_or(
                jnp.left_shift(jnp.right_shift(v, 14), 13),
                jnp.bitwise_and(loc, HALF - 1))
            is_hi = jnp.right_shift(loc, 13)
            mask_e = jnp.logical_and(is_hi == 0, valid)
            mask_o = jnp.logical_and(is_hi == 1, valid)
            ce = plsc.cumsum(jnp.where(mask_e, 1, 0).astype(jnp.int32))
            co = plsc.cumsum(jnp.where(mask_o, 1, 0).astype(jnp.int32))
            plsc.store_scatter(comb, [ne + ce - 1], row, mask=mask_e)
            plsc.store_scatter(comb, [jnp.int32(L - 1) - (no + co - 1)], row,
                               mask=mask_o)
            ne = ne + jnp.sum(jnp.where(mask_e, 1, 0).astype(jnp.int32))
            no = no + jnp.sum(jnp.where(mask_o, 1, 0).astype(jnp.int32))
        ne_s[r] = ne

    def fire(r, buf, sem, comb):
        compact(r, comb)
        pltpu.async_copy(
            emb_hbm.at[comb.at[pl.ds(0, CH0)]], buf.at[pl.ds(0, CH0)], sem)
        pltpu.async_copy(
            emb_hbm.at[comb.at[pl.ds(CH0, CH1)]], buf.at[pl.ds(CH0, CH1)], sem)

    def drain(buf, sem):
        pltpu.make_async_copy(emb_hbm.at[pl.ds(0, L)], buf, sem).wait()

    def reduce_row(buf, r):
        ne = ne_s[r]

        def phase(lo_start, lo_end, lane0, accs):
            def body(l, accs):
                a0, a1, a2, a3 = accs
                a0 = a0 + buf[l, pl.ds(lane0 + 0, 16)]
                a1 = a1 + buf[l, pl.ds(lane0 + 16, 16)]
                a2 = a2 + buf[l, pl.ds(lane0 + 32, 16)]
                a3 = a3 + buf[l, pl.ds(lane0 + 48, 16)]
                return a0, a1, a2, a3
            return lax.fori_loop(lo_start, lo_end, body, accs)

        z = jnp.zeros((16,), jnp.float32)
        accs = phase(0, ne, 0, (z, z, z, z))
        a0, a1, a2, a3 = phase(ne, L, EMB, accs)
        out_v[r, pl.ds(0, 16)] = a0
        out_v[r, pl.ds(16, 16)] = a1
        out_v[r, pl.ds(32, 16)] = a2
        out_v[r, pl.ds(48, 16)] = a3

    for b in range(NBUF):
        fire(b, bufs[b], sems[b], combs[b])

    def gbody(g, carry):
        for b in range(NBUF):
            r = g * NBUF + b
            drain(bufs[b], sems[b])
            reduce_row(bufs[b], r)
            nxt = r + NBUF

            @pl.when(nxt < ROWS_PER_W)
            def _():
                fire(nxt, bufs[b], sems[b], combs[b])
        return carry

    main = (ROWS_PER_W // NBUF) * NBUF
    lax.fori_loop(0, ROWS_PER_W // NBUF, gbody, 0)
    for r in range(main, ROWS_PER_W):
        b = r % NBUF
        drain(bufs[b], sems[b])
        reduce_row(bufs[b], r)
    pltpu.sync_copy(out_v, out_hbm.at[pl.ds(base, ROWS_PER_W)])


_sc_pool = functools.partial(
    pl.kernel,
    mesh=plsc.VectorSubcoreMesh(core_axis_name="c", subcore_axis_name="s"),
    compiler_params=pltpu.CompilerParams(
        use_tc_tiling_on_sc=False, needs_layout_passes=False),
    out_type=jax.ShapeDtypeStruct((B, EMB), jnp.float32),
    scratch_types=[
        pltpu.VMEM((ROWS_PER_W, L), jnp.int32),
        pltpu.VMEM((L, WIDE), jnp.float32),
        pltpu.VMEM((L, WIDE), jnp.float32),
        pltpu.VMEM((L, WIDE), jnp.float32),
        pltpu.VMEM((L,), jnp.int32),
        pltpu.VMEM((L,), jnp.int32),
        pltpu.VMEM((L,), jnp.int32),
        pltpu.SMEM((ROWS_PER_W,), jnp.int32),
        pltpu.VMEM((ROWS_PER_W, EMB), jnp.float32),
        pltpu.SemaphoreType.DMA,
        pltpu.SemaphoreType.DMA,
        pltpu.SemaphoreType.DMA,
    ],
)(_sc_pool_kernel)


def _mlp_kernel(p_ref, w1_ref, b1_ref, w2_ref, b2_ref, o_ref):
    p = p_ref[...] * (1.0 / L)
    h = jnp.dot(p, w1_ref[...], preferred_element_type=jnp.float32) + b1_ref[...]
    h = jnp.maximum(h, 0.0)
    o_ref[...] = jnp.dot(h, w2_ref[...], preferred_element_type=jnp.float32) + b2_ref[...]


def kernel(x, emb, W1, b1, W2, b2):
    table = _pack_table(emb.T)
    pooled_sum = _sc_pool(x, table)
    out = pl.pallas_call(
        _mlp_kernel,
        out_shape=jax.ShapeDtypeStruct((B, NCLS), jnp.float32),
    )(pooled_sum, W1, b1.reshape(1, HID), W2, b2.reshape(1, NCLS))
    return out


# speedup vs baseline: 2.4990x; 1.0503x over previous
import functools

import jax
import jax.numpy as jnp
from jax import lax
from jax.experimental import pallas as pl
from jax.experimental.pallas import tpu as pltpu
from jax.experimental.pallas import tpu_sc as plsc

VOCAB = 1000000
EMB = 64
HID = 128
NCLS = 50
B = 4096
L = 200
WIDE = 128

VBLK = 32768
HALF = VBLK // 2
NBLK = (VOCAB + VBLK - 1) // VBLK
TROWS = NBLK * HALF

NC = 2
NS = 16
NW = NC * NS
ROWS_PER_W = B // NW
CH0, CH1 = 128, L - 128
NBUF = 3
NCHUNK = 13


def _pack_kernel(in_ref, o_ref):
    row_i = lax.broadcasted_iota(jnp.int32, (EMB, WIDE), 0)
    col_i = lax.broadcasted_iota(jnp.int32, (EMB, WIDE), 1)
    eye_lo = (row_i == col_i).astype(jnp.float32)
    eye_hi = (row_i + EMB == col_i).astype(jnp.float32)
    dn = (((0,), (0,)), ((), ()))
    t0 = lax.dot_general(in_ref[:, 0:HALF], eye_lo, dn,
                         preferred_element_type=jnp.float32)
    t1 = lax.dot_general(in_ref[:, HALF:VBLK], eye_hi, dn,
                         preferred_element_type=jnp.float32)
    o_ref[...] = t0 + t1


def _pack_table(embT):
    return pl.pallas_call(
        _pack_kernel,
        grid=(NBLK,),
        in_specs=[pl.BlockSpec((EMB, VBLK), lambda i: (0, i))],
        out_specs=pl.BlockSpec((HALF, WIDE), lambda i: (i, 0)),
        out_shape=jax.ShapeDtypeStruct((TROWS, WIDE), jnp.float32),
    )(embT)


def _sc_pool_kernel(x_hbm, emb_hbm, out_hbm,
                    idx_v, b0, b1, b2, c0, c1, c2, ne_s, out_v, s0, s1, s2):
    bufs = (b0, b1, b2)
    combs = (c0, c1, c2)
    sems = (s0, s1, s2)
    wid = lax.axis_index("s") * NC + lax.axis_index("c")
    base = wid * ROWS_PER_W
    pltpu.sync_copy(x_hbm.at[pl.ds(base, ROWS_PER_W)], idx_v)

    lane = lax.iota(jnp.int32, 16)

    def compact(r, comb):
        ne = jnp.int32(0)
        no = jnp.int32(0)
        for c in range(NCHUNK):
            off = 184 if c == NCHUNK - 1 else c * 16
            v = idx_v[r, pl.ds(off, 16)]
            valid = lane >= 8 if c == NCHUNK - 1 else lane >= 0
            loc = jnp.bitwise_and(v, VBLK - 1)
            row = jnp.bitwise_or(
                jnp.left_shift(jnp.right_shift(v, 15), 14),
                jnp.bitwise_and(loc, HALF - 1))
            is_hi = jnp.right_shift(loc, 14)
            mask_e = jnp.logical_and(is_hi == 0, valid)
            mask_o = jnp.logical_and(is_hi == 1, valid)
            ce = plsc.cumsum(jnp.where(mask_e, 1, 0).astype(jnp.int32))
            co = plsc.cumsum(jnp.where(mask_o, 1, 0).astype(jnp.int32))
            plsc.store_scatter(comb, [ne + ce - 1], row, mask=mask_e)
            plsc.store_scatter(comb, [jnp.int32(L - 1) - (no + co - 1)], row,
                               mask=mask_o)
            ne = ne + jnp.sum(jnp.where(mask_e, 1, 0).astype(jnp.int32))
            no = no + jnp.sum(jnp.where(mask_o, 1, 0).astype(jnp.int32))
        ne_s[r] = ne

    def fire(r, buf, sem, comb):
        compact(r, comb)
        pltpu.async_copy(
            emb_hbm.at[comb.at[pl.ds(0, CH0)]], buf.at[pl.ds(0, CH0)], sem)
        pltpu.async_copy(
            emb_hbm.at[comb.at[pl.ds(CH0, CH1)]], buf.at[pl.ds(CH0, CH1)], sem)

    def drain(buf, sem):
        pltpu.make_async_copy(emb_hbm.at[pl.ds(0, L)], buf, sem).wait()

    def reduce_row(buf, r):
        ne = ne_s[r]

        def phase(lo_start, lo_end, lane0, accs):
            def body(l, accs):
                a0, a1, a2, a3 = accs
                a0 = a0 + buf[l, pl.ds(lane0 + 0, 16)]
                a1 = a1 + buf[l, pl.ds(lane0 + 16, 16)]
                a2 = a2 + buf[l, pl.ds(lane0 + 32, 16)]
                a3 = a3 + buf[l, pl.ds(lane0 + 48, 16)]
                return a0, a1, a2, a3
            return lax.fori_loop(lo_start, lo_end, body, accs)

        z = jnp.zeros((16,), jnp.float32)
        accs = phase(0, ne, 0, (z, z, z, z))
        a0, a1, a2, a3 = phase(ne, L, EMB, accs)
        out_v[r, pl.ds(0, 16)] = a0
        out_v[r, pl.ds(16, 16)] = a1
        out_v[r, pl.ds(32, 16)] = a2
        out_v[r, pl.ds(48, 16)] = a3

    for b in range(NBUF):
        fire(b, bufs[b], sems[b], combs[b])

    def gbody(g, carry):
        for b in range(NBUF):
            r = g * NBUF + b
            drain(bufs[b], sems[b])
            reduce_row(bufs[b], r)
            nxt = r + NBUF

            @pl.when(nxt < ROWS_PER_W)
            def _():
                fire(nxt, bufs[b], sems[b], combs[b])
        return carry

    main = (ROWS_PER_W // NBUF) * NBUF
    lax.fori_loop(0, ROWS_PER_W // NBUF, gbody, 0)
    for r in range(main, ROWS_PER_W):
        b = r % NBUF
        drain(bufs[b], sems[b])
        reduce_row(bufs[b], r)
    pltpu.sync_copy(out_v, out_hbm.at[pl.ds(base, ROWS_PER_W)])


_sc_pool = functools.partial(
    pl.kernel,
    mesh=plsc.VectorSubcoreMesh(core_axis_name="c", subcore_axis_name="s"),
    compiler_params=pltpu.CompilerParams(
        use_tc_tiling_on_sc=False, needs_layout_passes=False),
    out_type=jax.ShapeDtypeStruct((B, EMB), jnp.float32),
    scratch_types=[
        pltpu.VMEM((ROWS_PER_W, L), jnp.int32),
        pltpu.VMEM((L, WIDE), jnp.float32),
        pltpu.VMEM((L, WIDE), jnp.float32),
        pltpu.VMEM((L, WIDE), jnp.float32),
        pltpu.VMEM((L,), jnp.int32),
        pltpu.VMEM((L,), jnp.int32),
        pltpu.VMEM((L,), jnp.int32),
        pltpu.SMEM((ROWS_PER_W,), jnp.int32),
        pltpu.VMEM((ROWS_PER_W, EMB), jnp.float32),
        pltpu.SemaphoreType.DMA,
        pltpu.SemaphoreType.DMA,
        pltpu.SemaphoreType.DMA,
    ],
)(_sc_pool_kernel)


def _mlp_kernel(p_ref, w1_ref, b1_ref, w2_ref, b2_ref, o_ref):
    p = p_ref[...] * (1.0 / L)
    h = jnp.dot(p, w1_ref[...], preferred_element_type=jnp.float32) + b1_ref[...]
    h = jnp.maximum(h, 0.0)
    o_ref[...] = jnp.dot(h, w2_ref[...], preferred_element_type=jnp.float32) + b2_ref[...]


def kernel(x, emb, W1, b1, W2, b2):
    table = _pack_table(emb.T)
    pooled_sum = _sc_pool(x, table)
    out = pl.pallas_call(
        _mlp_kernel,
        out_shape=jax.ShapeDtypeStruct((B, NCLS), jnp.float32),
    )(pooled_sum, W1, b1.reshape(1, HID), W2, b2.reshape(1, NCLS))
    return out
